# Initial kernel scaffold; baseline (speedup 1.0000x reference)
import jax
import jax.numpy as jnp
from jax.experimental import pallas as pl


def kernel(coords_world, feat_maps):
    raise NotImplementedError("write your pallas kernel here")


# speedup vs baseline: 21.9139x; 21.9139x over previous
import functools
import math

import jax
import jax.numpy as jnp
from jax import lax
from jax.experimental import pallas as pl
from jax.experimental.pallas import tpu as pltpu
from jax.experimental.pallas import tpu_sc as plsc

_ANGLES = (-90.0, -60.0, -30.0, 0.0, 30.0, 60.0, 90.0)
_B = 2
_N = 65536
_V = 7
_C = 32
_HW = 64
_PIX = _HW * _HW
_HALF_FOV = 40.0
_P = 64
_PT = _N // 16
_NCHUNK = _PT // _P


_GDN = lax.GatherDimensionNumbers(
    offset_dims=(), collapsed_slice_dims=(0,), start_index_map=(0,))


def _vsplat(vec, i):
    idx = jnp.full((16, 1), i, jnp.int32)
    return lax.gather(vec, idx, _GDN, (1,),
                      mode=lax.GatherScatterMode.PROMISE_IN_BOUNDS)


def _sc_body(coords, feat, outf, vis, xb, yb, zb,
             i00b, i10b, i01b, i11b, w00b, w10b, w01b, w11b,
             g00, g10, g01, g11, stage, visst, sem):
    b = lax.axis_index("c")
    s = lax.axis_index("s")

    lane = lax.iota(jnp.int32, 16)
    cbase = b * 3 * _N
    tboff = b * (_V * _PIX)
    tbase = s * _PT

    def chunk_body(ci, carry):
        n0 = tbase + ci * _P
        pltpu.sync_copy(coords.at[pl.ds(cbase + n0, _P)], xb)
        pltpu.sync_copy(coords.at[pl.ds(cbase + _N + n0, _P)], yb)
        pltpu.sync_copy(coords.at[pl.ds(cbase + 2 * _N + n0, _P)], zb)

        for v in range(_V):
            a = math.radians(_ANGLES[v])
            cos_v = jnp.float32(math.cos(a))
            sin_v = jnp.float32(math.sin(a))

            def proj_body(g, carry2):
                sl = pl.ds(g * 16, 16)
                x = xb[sl]
                y = yb[sl]
                z = zb[sl]
                x_rot = x * cos_v + z * sin_v
                u = x_rot / _HALF_FOV
                w = (y - 30.0) / _HALF_FOV
                valid = (jnp.abs(u) <= 1.0) & (jnp.abs(w) <= 1.0)
                ix = ((u + 1.0) * 64.0 - 1.0) * 0.5
                iy = ((w + 1.0) * 64.0 - 1.0) * 0.5
                zf = jnp.zeros((16,), jnp.float32)
                onei = jnp.ones((16,), jnp.int32)
                zi = jnp.zeros((16,), jnp.int32)
                xt = ix.astype(jnp.int32)
                xtf = xt.astype(jnp.float32)
                x0 = jnp.where(ix < xtf, xt - onei, xt)
                yt = iy.astype(jnp.int32)
                ytf = yt.astype(jnp.float32)
                y0 = jnp.where(iy < ytf, yt - onei, yt)
                x0f = jnp.where(ix < xtf, xtf - 1.0, xtf)
                y0f = jnp.where(iy < ytf, ytf - 1.0, ytf)
                wx1 = ix - x0f
                wx0 = 1.0 - wx1
                wy1 = iy - y0f
                wy0 = 1.0 - wy1
                wx0e = jnp.where(x0 >= 0, wx0, zf)
                wx1e = jnp.where(x0 <= 62, wx1, zf)
                wy0m = jnp.where(y0 >= 0, wy0, zf)
                wy1m = jnp.where(y0 <= 62, wy1, zf)
                wy0e = jnp.where(valid, wy0m, zf)
                wy1e = jnp.where(valid, wy1m, zf)
                w00b[sl] = wx0e * wy0e
                w10b[sl] = wx1e * wy0e
                w01b[sl] = wx0e * wy1e
                w11b[sl] = wx1e * wy1e
                xc0 = jnp.clip(x0, 0, 63)
                xc1 = jnp.clip(x0 + 1, 0, 63)
                yc0 = jnp.clip(y0, 0, 63)
                yc1 = jnp.clip(y0 + 1, 0, 63)
                row0 = yc0 * 64 + (v * _PIX + tboff)
                row1 = yc1 * 64 + (v * _PIX + tboff)
                i00b[sl] = row0 + xc0
                i10b[sl] = row0 + xc1
                i01b[sl] = row1 + xc0
                i11b[sl] = row1 + xc1
                rowvec = g * 16 + lane
                plsc.store_scatter(visst, [rowvec * _V + v],
                                   jnp.where(valid, onei, zi))
                return carry2

            lax.fori_loop(0, _P // 16, proj_body, 0)

            pltpu.async_copy(feat.at[i00b], g00, sem).wait()
            pltpu.async_copy(feat.at[i10b], g10, sem).wait()
            pltpu.async_copy(feat.at[i01b], g01, sem).wait()
            pltpu.async_copy(feat.at[i11b], g11, sem).wait()

            def comb_body(g, carry2):
                sl = pl.ds(g * 16, 16)
                w00v = w00b[sl]
                w10v = w10b[sl]
                w01v = w01b[sl]
                w11v = w11b[sl]
                for rl in range(16):
                    r = g * 16 + rl
                    s00 = _vsplat(w00v, rl)
                    s10 = _vsplat(w10v, rl)
                    s01 = _vsplat(w01v, rl)
                    s11 = _vsplat(w11v, rl)
                    for hh in range(2):
                        cs = pl.ds(hh * 16, 16)
                        acc = (g00[r, cs] * s00 + g10[r, cs] * s10
                               + g01[r, cs] * s01 + g11[r, cs] * s11)
                        stage[pl.ds(r * (_V * _C) + v * _C + hh * 16, 16)] = acc
                return carry2

            lax.fori_loop(0, _P // 16, comb_body, 0)

        pltpu.sync_copy(stage,
                        outf.at[pl.ds((b * _N + n0) * (_V * _C),
                                      _P * _V * _C)])
        pltpu.sync_copy(visst, vis.at[pl.ds((b * _N + n0) * _V, _P * _V)])
        return carry

    lax.fori_loop(0, _NCHUNK, chunk_body, 0)


_mesh = plsc.VectorSubcoreMesh(core_axis_name="c", subcore_axis_name="s")

_sc_call = functools.partial(
    pl.kernel,
    out_type=[
        jax.ShapeDtypeStruct((_B * _N * _V * _C,), jnp.float32),
        jax.ShapeDtypeStruct((_B * _N * _V,), jnp.int32),
    ],
    mesh=_mesh,
    compiler_params=pltpu.CompilerParams(needs_layout_passes=False,
                                         use_tc_tiling_on_sc=False),
    scratch_types=[
        pltpu.VMEM((_P,), jnp.float32),
        pltpu.VMEM((_P,), jnp.float32),
        pltpu.VMEM((_P,), jnp.float32),
        pltpu.VMEM((_P,), jnp.int32),
        pltpu.VMEM((_P,), jnp.int32),
        pltpu.VMEM((_P,), jnp.int32),
        pltpu.VMEM((_P,), jnp.int32),
        pltpu.VMEM((_P,), jnp.float32),
        pltpu.VMEM((_P,), jnp.float32),
        pltpu.VMEM((_P,), jnp.float32),
        pltpu.VMEM((_P,), jnp.float32),
        pltpu.VMEM((_P, _C), jnp.float32),
        pltpu.VMEM((_P, _C), jnp.float32),
        pltpu.VMEM((_P, _C), jnp.float32),
        pltpu.VMEM((_P, _C), jnp.float32),
        pltpu.VMEM((_P * _V * _C,), jnp.float32),
        pltpu.VMEM((_P * _V,), jnp.int32),
        pltpu.SemaphoreType.DMA,
    ],
)(_sc_body)


def kernel(coords_world, feat_maps):
    coords_t = jnp.transpose(coords_world, (0, 2, 1)).reshape(-1)
    feat_r = jnp.transpose(feat_maps, (0, 1, 3, 4, 2))
    feat_r = feat_r.reshape(_B * _V * _PIX, _C)
    outf, vis = _sc_call(coords_t, feat_r)
    multi_view_feat = outf.reshape(_B, _N, _V, _C)
    visibility = vis.reshape(_B, _N, _V) != 0
    return multi_view_feat, visibility


# speedup vs baseline: 36.6768x; 1.6737x over previous
import functools
import math

import jax
import jax.numpy as jnp
from jax import lax
from jax.experimental import pallas as pl
from jax.experimental.pallas import tpu as pltpu
from jax.experimental.pallas import tpu_sc as plsc

_ANGLES = (-90.0, -60.0, -30.0, 0.0, 30.0, 60.0, 90.0)
_B = 2
_N = 65536
_V = 7
_C = 32
_HW = 64
_PIX = _HW * _HW
_HALF_FOV = 40.0
_P = 64
_PT = _N // 16
_NCHUNK = _PT // _P


_GDN = lax.GatherDimensionNumbers(
    offset_dims=(), collapsed_slice_dims=(0,), start_index_map=(0,))


def _vsplat(vec, i):
    idx = jnp.full((16, 1), i, jnp.int32)
    return lax.gather(vec, idx, _GDN, (1,),
                      mode=lax.GatherScatterMode.PROMISE_IN_BOUNDS)


def _sc_body(coords, feat, outf, vis, xb, yb, zb,
             i00b, i10b, i01b, i11b, w00b, w10b, w01b, w11b,
             stage, visst, *rest):
    gbs = rest[:4 * _V]
    sem = rest[4 * _V]
    b = lax.axis_index("c")
    s = lax.axis_index("s")

    lane = lax.iota(jnp.int32, 16)
    cbase = b * 3 * _N
    tboff = b * (_V * _PIX)
    tbase = s * _PT

    def chunk_body(ci, carry):
        n0 = tbase + ci * _P
        pltpu.sync_copy(coords.at[pl.ds(cbase + n0, _P)], xb)
        pltpu.sync_copy(coords.at[pl.ds(cbase + _N + n0, _P)], yb)
        pltpu.sync_copy(coords.at[pl.ds(cbase + 2 * _N + n0, _P)], zb)

        for v in range(_V):
            a = math.radians(_ANGLES[v])
            cos_v = jnp.float32(math.cos(a))
            sin_v = jnp.float32(math.sin(a))

            def proj_body(g, carry2):
                sl = pl.ds(g * 16, 16)
                sl7 = pl.ds(v * _P + g * 16, 16)
                x = xb[sl]
                y = yb[sl]
                z = zb[sl]
                x_rot = x * cos_v + z * sin_v
                u = x_rot / _HALF_FOV
                w = (y - 30.0) / _HALF_FOV
                valid = (jnp.abs(u) <= 1.0) & (jnp.abs(w) <= 1.0)
                ix = ((u + 1.0) * 64.0 - 1.0) * 0.5
                iy = ((w + 1.0) * 64.0 - 1.0) * 0.5
                zf = jnp.zeros((16,), jnp.float32)
                onei = jnp.ones((16,), jnp.int32)
                zi = jnp.zeros((16,), jnp.int32)
                xt = ix.astype(jnp.int32)
                xtf = xt.astype(jnp.float32)
                x0 = jnp.where(ix < xtf, xt - onei, xt)
                yt = iy.astype(jnp.int32)
                ytf = yt.astype(jnp.float32)
                y0 = jnp.where(iy < ytf, yt - onei, yt)
                x0f = jnp.where(ix < xtf, xtf - 1.0, xtf)
                y0f = jnp.where(iy < ytf, ytf - 1.0, ytf)
                wx1 = ix - x0f
                wx0 = 1.0 - wx1
                wy1 = iy - y0f
                wy0 = 1.0 - wy1
                wx0e = jnp.where(x0 >= 0, wx0, zf)
                wx1e = jnp.where(x0 <= 62, wx1, zf)
                wy0m = jnp.where(y0 >= 0, wy0, zf)
                wy1m = jnp.where(y0 <= 62, wy1, zf)
                wy0e = jnp.where(valid, wy0m, zf)
                wy1e = jnp.where(valid, wy1m, zf)
                w00b[sl7] = wx0e * wy0e
                w10b[sl7] = wx1e * wy0e
                w01b[sl7] = wx0e * wy1e
                w11b[sl7] = wx1e * wy1e
                xc0 = jnp.clip(x0, 0, 63)
                xc1 = jnp.clip(x0 + 1, 0, 63)
                yc0 = jnp.clip(y0, 0, 63)
                yc1 = jnp.clip(y0 + 1, 0, 63)
                row0 = yc0 * 64 + (v * _PIX + tboff)
                row1 = yc1 * 64 + (v * _PIX + tboff)
                i00b[sl7] = row0 + xc0
                i10b[sl7] = row0 + xc1
                i01b[sl7] = row1 + xc0
                i11b[sl7] = row1 + xc1
                rowvec = g * 16 + lane
                plsc.store_scatter(visst, [rowvec * _V + v],
                                   jnp.where(valid, onei, zi))
                return carry2

            lax.fori_loop(0, _P // 16, proj_body, 0)

        copies = []
        for v in range(_V):
            vsl = pl.ds(v * _P, _P)
            copies.append(pltpu.async_copy(
                feat.at[i00b.at[vsl]], gbs[4 * v + 0], sem))
            copies.append(pltpu.async_copy(
                feat.at[i10b.at[vsl]], gbs[4 * v + 1], sem))
            copies.append(pltpu.async_copy(
                feat.at[i01b.at[vsl]], gbs[4 * v + 2], sem))
            copies.append(pltpu.async_copy(
                feat.at[i11b.at[vsl]], gbs[4 * v + 3], sem))
        for c in copies:
            c.wait()

        for v in range(_V):
            g00 = gbs[4 * v + 0]
            g10 = gbs[4 * v + 1]
            g01 = gbs[4 * v + 2]
            g11 = gbs[4 * v + 3]

            def comb_body(g, carry2, g00=g00, g10=g10, g01=g01, g11=g11, v=v):
                sl7 = pl.ds(v * _P + g * 16, 16)
                w00v = w00b[sl7]
                w10v = w10b[sl7]
                w01v = w01b[sl7]
                w11v = w11b[sl7]
                for rl in range(16):
                    r = g * 16 + rl
                    s00 = _vsplat(w00v, rl)
                    s10 = _vsplat(w10v, rl)
                    s01 = _vsplat(w01v, rl)
                    s11 = _vsplat(w11v, rl)
                    for hh in range(2):
                        cs = pl.ds(hh * 16, 16)
                        acc = (g00[r, cs] * s00 + g10[r, cs] * s10
                               + g01[r, cs] * s01 + g11[r, cs] * s11)
                        stage[pl.ds(r * (_V * _C) + v * _C + hh * 16, 16)] = acc
                return carry2

            lax.fori_loop(0, _P // 16, comb_body, 0)

        pltpu.sync_copy(stage,
                        outf.at[pl.ds((b * _N + n0) * (_V * _C),
                                      _P * _V * _C)])
        pltpu.sync_copy(visst, vis.at[pl.ds((b * _N + n0) * _V, _P * _V)])
        return carry

    lax.fori_loop(0, _NCHUNK, chunk_body, 0)


_mesh = plsc.VectorSubcoreMesh(core_axis_name="c", subcore_axis_name="s")

_sc_call = functools.partial(
    pl.kernel,
    out_type=[
        jax.ShapeDtypeStruct((_B * _N * _V * _C,), jnp.float32),
        jax.ShapeDtypeStruct((_B * _N * _V,), jnp.int32),
    ],
    mesh=_mesh,
    compiler_params=pltpu.CompilerParams(needs_layout_passes=False,
                                         use_tc_tiling_on_sc=False),
    scratch_types=[
        pltpu.VMEM((_P,), jnp.float32),
        pltpu.VMEM((_P,), jnp.float32),
        pltpu.VMEM((_P,), jnp.float32),
        pltpu.VMEM((_V * _P,), jnp.int32),
        pltpu.VMEM((_V * _P,), jnp.int32),
        pltpu.VMEM((_V * _P,), jnp.int32),
        pltpu.VMEM((_V * _P,), jnp.int32),
        pltpu.VMEM((_V * _P,), jnp.float32),
        pltpu.VMEM((_V * _P,), jnp.float32),
        pltpu.VMEM((_V * _P,), jnp.float32),
        pltpu.VMEM((_V * _P,), jnp.float32),
        pltpu.VMEM((_P * _V * _C,), jnp.float32),
        pltpu.VMEM((_P * _V,), jnp.int32),
    ] + [pltpu.VMEM((_P, _C), jnp.float32) for _ in range(4 * _V)]
      + [pltpu.SemaphoreType.DMA],
)(_sc_body)


def kernel(coords_world, feat_maps):
    coords_t = jnp.transpose(coords_world, (0, 2, 1)).reshape(-1)
    feat_r = jnp.transpose(feat_maps, (0, 1, 3, 4, 2))
    feat_r = feat_r.reshape(_B * _V * _PIX, _C)
    outf, vis = _sc_call(coords_t, feat_r)
    multi_view_feat = outf.reshape(_B, _N, _V, _C)
    visibility = vis.reshape(_B, _N, _V) != 0
    return multi_view_feat, visibility
